# baseline (speedup 1.0000x reference)
import jax
import jax.numpy as jnp
from jax.experimental import pallas as pl
from jax.experimental.pallas import tpu as pltpu

N_DIAG = 1000
N_PROC = 1000
N_MED = 300
D = 128
LEVELS = 5
NUM_REL = 2 * LEVELS + 2
M_PAD = 384


def _rgcn2_kernel(ed_hbm, ep_hbm, em_hbm, dw_hbm, pw_hbm, w1_hbm, w2_hbm,
                  r1_ref, b1_ref, r2_ref, b2_ref,
                  od_ref, op_ref, om_ref,
                  ed_v, ep_v, em_v, dw_v, pw_v, w1_v, w2_v,
                  s_ed, s_ep, s_em, s_dw, s_pw, s_w1, s_w2):
    cp_dw = pltpu.make_async_copy(dw_hbm, dw_v, s_dw)
    cp_pw = pltpu.make_async_copy(pw_hbm, pw_v, s_pw)
    cp_em = pltpu.make_async_copy(em_hbm, em_v, s_em)
    cp_w1 = pltpu.make_async_copy(w1_hbm, w1_v, s_w1)
    cp_ed = pltpu.make_async_copy(ed_hbm, ed_v, s_ed)
    cp_ep = pltpu.make_async_copy(ep_hbm, ep_v, s_ep)
    cp_w2 = pltpu.make_async_copy(w2_hbm, w2_v, s_w2)
    for cp in (cp_dw, cp_pw, cp_em, cp_w1, cp_ed, cp_ep, cp_w2):
        cp.start()

    def a_cat(w, iz):
        parts = []
        for i in range(1, LEVELS + 1):
            m = jnp.where((w > i / LEVELS) & (w <= (i + 1) / LEVELS),
                          1.0 - iz, 0.0)
            c = jnp.sum(m, axis=1, keepdims=True)
            parts.append(m / jnp.maximum(c, 1.0))
        return jnp.concatenate(parts, axis=1)

    cp_dw.wait()
    dw = dw_v[...]
    izd = jnp.min(jnp.where(dw == 0.0, 1.0, 0.0))
    ad = a_cat(dw, izd)
    cp_pw.wait()
    pw = pw_v[...]
    izp = jnp.min(jnp.where(pw == 0.0, 1.0, 0.0))
    ap = a_cat(pw, izp)

    def layer(xd, xp, xm, w, root, bias):
        hd = jnp.concatenate(
            [jnp.dot(xm, w[1 + i], preferred_element_type=jnp.float32)
             for i in range(LEVELS)], axis=0)
        hp = jnp.concatenate(
            [jnp.dot(xm, w[LEVELS + 2 + i], preferred_element_type=jnp.float32)
             for i in range(LEVELS)], axis=0)
        agg_d = jnp.dot(ad, hd, preferred_element_type=jnp.float32)
        agg_p = jnp.dot(ap, hp, preferred_element_type=jnp.float32)
        h0 = izd * jnp.dot(xm[0:1], w[0], preferred_element_type=jnp.float32)
        h6 = izp * jnp.dot(xm[0:1], w[LEVELS + 1],
                           preferred_element_type=jnp.float32)
        out_d = agg_d + h0 + bias + jnp.dot(
            xd, root, preferred_element_type=jnp.float32)
        out_p = agg_p + h0 + h6 + bias + jnp.dot(
            xp, root, preferred_element_type=jnp.float32)
        out_m = h0 + h6 + bias + jnp.dot(
            xm, root, preferred_element_type=jnp.float32)
        return out_d, out_p, out_m

    cp_em.wait()
    cp_w1.wait()
    cp_ed.wait()
    cp_ep.wait()
    ed = ed_v[0]
    ep = ep_v[0]
    xm1 = em_v[...]
    d1, p1, m1 = layer(ed, ep, xm1, w1_v[...], r1_ref[...], b1_ref[...])
    d1 = jax.nn.relu(d1)
    p1 = jax.nn.relu(p1)
    m1 = jax.nn.relu(m1)
    cp_w2.wait()
    d2, p2, m2 = layer(d1, p1, m1, w2_v[...], r2_ref[...], b2_ref[...])
    od_ref[0] = d2
    op_ref[0] = p2
    om_ref[0] = m2[:N_MED]


def _run(ed, ep, em, dw, pw, w1, w2, r1, b1, r2, b2, interpret=False):
    hbm = pl.BlockSpec(memory_space=pltpu.MemorySpace.HBM)
    vmem = pl.BlockSpec(memory_space=pltpu.MemorySpace.VMEM)
    return pl.pallas_call(
        _rgcn2_kernel,
        out_shape=(
            jax.ShapeDtypeStruct((1, N_DIAG, D), jnp.float32),
            jax.ShapeDtypeStruct((1, N_PROC, D), jnp.float32),
            jax.ShapeDtypeStruct((1, N_MED, D), jnp.float32),
        ),
        in_specs=[hbm, hbm, hbm, hbm, hbm, hbm, hbm,
                  vmem, vmem, vmem, vmem],
        scratch_shapes=[
            pltpu.VMEM((1, N_DIAG, D), jnp.float32),
            pltpu.VMEM((1, N_PROC, D), jnp.float32),
            pltpu.VMEM((M_PAD, D), jnp.float32),
            pltpu.VMEM((N_DIAG, M_PAD), jnp.float32),
            pltpu.VMEM((N_PROC, M_PAD), jnp.float32),
            pltpu.VMEM((NUM_REL, D, D), jnp.float32),
            pltpu.VMEM((NUM_REL, D, D), jnp.float32),
            pltpu.SemaphoreType.DMA,
            pltpu.SemaphoreType.DMA,
            pltpu.SemaphoreType.DMA,
            pltpu.SemaphoreType.DMA,
            pltpu.SemaphoreType.DMA,
            pltpu.SemaphoreType.DMA,
            pltpu.SemaphoreType.DMA,
        ],
        interpret=interpret,
    )(ed, ep, em, dw, pw, w1, w2, r1, b1, r2, b2)


@jax.jit
def kernel(emb_diag, emb_proc, emb_med, diag_med_weights, proc_med_weights,
           W1, root1, b1, W2, root2, b2):
    pad_m = M_PAD - N_MED
    em = jnp.pad(emb_med[0], ((0, pad_m), (0, 0)))
    dw = jnp.pad(diag_med_weights, ((0, 0), (0, pad_m)))
    pw = jnp.pad(proc_med_weights, ((0, 0), (0, pad_m)))
    return _run(emb_diag, emb_proc, em, dw, pw, W1, W2,
                root1, b1.reshape(1, D), root2, b2.reshape(1, D))


# speedup vs baseline: 1.0987x; 1.0987x over previous
import jax
import jax.numpy as jnp
from jax.experimental import pallas as pl

N_DIAG = 1000
N_PROC = 1000
N_MED = 300
D = 128
LEVELS = 5
M_PAD = 384


def _rgcn2_kernel(ed_ref, ep_ref, em_ref, dw_ref, pw_ref,
                  w1_ref, r1_ref, b1_ref, w2_ref, r2_ref, b2_ref,
                  od_ref, op_ref, om_ref):
    ed = ed_ref[0]
    ep = ep_ref[0]
    xm1 = em_ref[...]
    dw = dw_ref[...]
    pw = pw_ref[...]

    izd = jnp.min(jnp.where(dw == 0.0, 1.0, 0.0))
    izp = jnp.min(jnp.where(pw == 0.0, 1.0, 0.0))

    def a_cat(w, iz):
        parts = []
        for i in range(1, LEVELS + 1):
            m = jnp.where((w > i / LEVELS) & (w <= (i + 1) / LEVELS),
                          1.0 - iz, 0.0)
            c = jnp.sum(m, axis=1, keepdims=True)
            parts.append(m / jnp.maximum(c, 1.0))
        return jnp.concatenate(parts, axis=1)

    ad = a_cat(dw, izd)
    ap = a_cat(pw, izp)

    def layer(xd, xp, xm, w, root, bias):
        hd = jnp.concatenate(
            [jnp.dot(xm, w[1 + i], preferred_element_type=jnp.float32)
             for i in range(LEVELS)], axis=0)
        hp = jnp.concatenate(
            [jnp.dot(xm, w[LEVELS + 2 + i], preferred_element_type=jnp.float32)
             for i in range(LEVELS)], axis=0)
        agg_d = jnp.dot(ad, hd, preferred_element_type=jnp.float32)
        agg_p = jnp.dot(ap, hp, preferred_element_type=jnp.float32)
        h0 = izd * jnp.dot(xm[0:1], w[0], preferred_element_type=jnp.float32)
        h6 = izp * jnp.dot(xm[0:1], w[LEVELS + 1],
                           preferred_element_type=jnp.float32)
        out_d = agg_d + h0 + bias + jnp.dot(
            xd, root, preferred_element_type=jnp.float32)
        out_p = agg_p + h0 + h6 + bias + jnp.dot(
            xp, root, preferred_element_type=jnp.float32)
        out_m = h0 + h6 + bias + jnp.dot(
            xm, root, preferred_element_type=jnp.float32)
        return out_d, out_p, out_m

    d1, p1, m1 = layer(ed, ep, xm1, w1_ref[...], r1_ref[...], b1_ref[...])
    d1 = jax.nn.relu(d1)
    p1 = jax.nn.relu(p1)
    m1 = jax.nn.relu(m1)
    d2, p2, m2 = layer(d1, p1, m1, w2_ref[...], r2_ref[...], b2_ref[...])
    od_ref[0] = d2
    op_ref[0] = p2
    om_ref[0] = m2[:N_MED]


def _run(ed, ep, em, dw, pw, w1, r1, b1, w2, r2, b2, interpret=False):
    return pl.pallas_call(
        _rgcn2_kernel,
        out_shape=(
            jax.ShapeDtypeStruct((1, N_DIAG, D), jnp.float32),
            jax.ShapeDtypeStruct((1, N_PROC, D), jnp.float32),
            jax.ShapeDtypeStruct((1, N_MED, D), jnp.float32),
        ),
        interpret=interpret,
    )(ed, ep, em, dw, pw, w1, r1, b1, w2, r2, b2)


@jax.jit
def kernel(emb_diag, emb_proc, emb_med, diag_med_weights, proc_med_weights,
           W1, root1, b1, W2, root2, b2):
    pad_m = M_PAD - N_MED
    em = jnp.pad(emb_med[0], ((0, pad_m), (0, 0)))
    dw = jnp.pad(diag_med_weights, ((0, 0), (0, pad_m)))
    pw = jnp.pad(proc_med_weights, ((0, 0), (0, pad_m)))
    return _run(emb_diag, emb_proc, em, dw, pw,
                W1, root1, b1.reshape(1, D), W2, root2, b2.reshape(1, D))
